# baseline (device time: 20814 ns/iter reference)
import jax
import jax.numpy as jnp
from jax import lax
from jax.experimental import pallas as pl
from jax.experimental.pallas import tpu as pltpu

N_DEV = 16


def kernel(x, pi):
    _, m, n = x.shape

    def body(pi_ref, x_ref, out_ref, send_buf, send_sem, recv_sem):
        my = lax.axis_index("i")
        dst = pi_ref[my]
        src = lax.fori_loop(
            0,
            N_DEV,
            lambda j, acc: jnp.where(pi_ref[j] == my, j, acc),
            jnp.int32(0),
        )

        bar = pltpu.get_barrier_semaphore()
        pl.semaphore_signal(
            bar, inc=1, device_id=(src,), device_id_type=pl.DeviceIdType.MESH
        )

        send_buf[...] = x_ref[0].astype(jnp.bfloat16)

        pl.semaphore_wait(bar, 1)

        rdma = pltpu.make_async_remote_copy(
            src_ref=send_buf,
            dst_ref=out_ref.at[0],
            send_sem=send_sem,
            recv_sem=recv_sem,
            device_id=(dst,),
            device_id_type=pl.DeviceIdType.MESH,
        )
        rdma.start()
        rdma.wait_send()
        rdma.wait_recv()

    return pl.pallas_call(
        body,
        out_shape=jax.ShapeDtypeStruct((1, m, n), jnp.bfloat16),
        in_specs=[
            pl.BlockSpec(memory_space=pltpu.SMEM),
            pl.BlockSpec(memory_space=pltpu.VMEM),
        ],
        out_specs=pl.BlockSpec(memory_space=pltpu.VMEM),
        scratch_shapes=[
            pltpu.VMEM((m, n), jnp.bfloat16),
            pltpu.SemaphoreType.DMA,
            pltpu.SemaphoreType.DMA,
        ],
        compiler_params=pltpu.CompilerParams(collective_id=0),
    )(pi, x)
